# Initial kernel scaffold; baseline (speedup 1.0000x reference)
import jax
import jax.numpy as jnp
from jax.experimental import pallas as pl


def kernel(x):
    raise NotImplementedError("write your pallas kernel here")


# speedup vs baseline: 1.5460x; 1.5460x over previous
import functools

import jax
import jax.numpy as jnp
import numpy as np
from jax import lax
from jax.experimental import pallas as pl
from jax.experimental.pallas import tpu as pltpu
from jax.experimental.pallas import tpu_sc as plsc

ROWS = 64
N = 32768
LANES = 16
SEG = N // LANES
NBINS = 256
NPASS = 4
WORKERS = 32
ROWS_PER_WORKER = ROWS // WORKERS
INT_MIN = np.int32(-2147483648)


def _row_sort_body(key, ia, ib, hist):
  iota = lax.iota(jnp.int32, LANES)
  seg_base = iota * SEG
  lane_addr = iota
  ones = jnp.full((LANES,), 1, jnp.int32)

  def xform(t, _):
    sl = pl.ds(t * LANES, LANES)
    v = key[sl]
    flip = lax.shift_right_arithmetic(v, 31) | INT_MIN
    key[sl] = v ^ flip
    return _

  lax.fori_loop(0, N // LANES, xform, None)

  for p in range(NPASS):
    shift = 8 * p
    src = (ia, ib)[p % 2] if p > 0 else None
    dst = (ia, ib)[(p + 1) % 2]

    def zero(j, _):
      hist[pl.ds(j * LANES, LANES)] = jnp.zeros((LANES,), jnp.int32)
      return _

    lax.fori_loop(0, NBINS, zero, None)

    def count(t, _, shift=shift, src=src):
      pos = seg_base + t
      idxv = pos if src is None else plsc.load_gather(src, [pos])
      k = plsc.load_gather(key, [idxv])
      d = lax.shift_right_logical(k, shift) & 255
      plsc.addupdate_scatter(hist, [d * LANES + lane_addr], ones)
      return _

    lax.fori_loop(0, SEG, count, None)

    def scan(j, carry):
      sl = pl.ds(j * LANES, LANES)
      v = hist[sl]
      cs = plsc.cumsum(v)
      hist[sl] = cs - v + carry
      return carry + jnp.max(cs)

    lax.fori_loop(0, NBINS, scan, jnp.int32(0))

    def permute(t, _, shift=shift, src=src, dst=dst):
      pos = seg_base + t
      idxv = pos if src is None else plsc.load_gather(src, [pos])
      k = plsc.load_gather(key, [idxv])
      d = lax.shift_right_logical(k, shift) & 255
      addr = d * LANES + lane_addr
      base = plsc.load_gather(hist, [addr])
      plsc.store_scatter(dst, [base], idxv)
      plsc.addupdate_scatter(hist, [addr], ones)
      return _

    lax.fori_loop(0, SEG, permute, None)

  def emit(t, _):
    sl = pl.ds(t * LANES, LANES)
    idxv = ia[sl]
    k = plsc.load_gather(key, [idxv])
    flip = (~lax.shift_right_arithmetic(k, 31)) | INT_MIN
    ib[sl] = k ^ flip
    return _

  lax.fori_loop(0, N // LANES, emit, None)


@functools.cache
def _make_sort_kernel():
  mesh = plsc.VectorSubcoreMesh(core_axis_name="c", subcore_axis_name="s")

  @functools.partial(
      pl.kernel,
      out_type=(
          jax.ShapeDtypeStruct((ROWS, N), jnp.int32),
          jax.ShapeDtypeStruct((ROWS, N), jnp.int32),
      ),
      mesh=mesh,
      compiler_params=pltpu.CompilerParams(needs_layout_passes=False),
      scratch_types=[
          pltpu.VMEM((N,), jnp.int32),
          pltpu.VMEM((N,), jnp.int32),
          pltpu.VMEM((N,), jnp.int32),
          pltpu.VMEM((NBINS * LANES,), jnp.int32),
      ],
  )
  def sort_kernel(x_hbm, vals_hbm, idx_hbm, key, ia, ib, hist):
    wid = lax.axis_index("s") * 2 + lax.axis_index("c")
    for i in range(ROWS_PER_WORKER):
      r = wid * ROWS_PER_WORKER + i
      pltpu.sync_copy(x_hbm.at[r], key)
      _row_sort_body(key, ia, ib, hist)
      pltpu.sync_copy(ib, vals_hbm.at[r])
      pltpu.sync_copy(ia, idx_hbm.at[r])

  return sort_kernel


def kernel(x):
  bits = lax.bitcast_convert_type(x, jnp.int32)
  vals_bits, idx = _make_sort_kernel()(bits)
  vals = lax.bitcast_convert_type(vals_bits, jnp.float32)
  ok = jnp.array(True)
  return vals, idx, ok


# speedup vs baseline: 1.5864x; 1.0262x over previous
import functools

import jax
import jax.numpy as jnp
import numpy as np
from jax import lax
from jax.experimental import pallas as pl
from jax.experimental.pallas import tpu as pltpu
from jax.experimental.pallas import tpu_sc as plsc

ROWS = 64
N = 32768
LANES = 16
SEG = N // LANES
NBINS = 256
NPASS = 4
WORKERS = 32
ROWS_PER_WORKER = ROWS // WORKERS
INT_MIN = np.int32(-2147483648)
UNROLL = 8


def _row_sort_body(key, ia, ib, hist):
  iota = lax.iota(jnp.int32, LANES)
  seg_base = iota * SEG
  ones = jnp.full((LANES,), 1, jnp.int32)
  zeros = jnp.zeros((LANES,), jnp.int32)

  def zero_hist(j, _):
    hist[pl.ds(j * LANES, LANES)] = zeros
    return _

  lax.fori_loop(0, NBINS, zero_hist, None, unroll=UNROLL)

  def xform_count(t, _):
    pos = seg_base + t
    v = plsc.load_gather(key, [pos])
    flip = lax.shift_right_arithmetic(v, 31) | INT_MIN
    k = v ^ flip
    plsc.store_scatter(key, [pos], k)
    d = k & 255
    plsc.addupdate_scatter(hist, [d * LANES + iota], ones)
    return _

  lax.fori_loop(0, SEG, xform_count, None, unroll=UNROLL)

  for p in range(NPASS):
    shift = 8 * p
    src = (ia, ib)[p % 2] if p > 0 else None
    dst = (ia, ib)[(p + 1) % 2]

    if p > 0:

      def count(t, _, shift=shift, src=src):
        pos = seg_base + t
        idxv = plsc.load_gather(src, [pos])
        k = plsc.load_gather(key, [idxv])
        d = lax.shift_right_logical(k, shift) & 255
        plsc.addupdate_scatter(hist, [d * LANES + iota], ones)
        return _

      lax.fori_loop(0, NBINS, zero_hist, None, unroll=UNROLL)
      lax.fori_loop(0, SEG, count, None, unroll=UNROLL)

    def scan(j, carry):
      sl = pl.ds(j * LANES, LANES)
      v = hist[sl]
      cs = plsc.cumsum(v)
      hist[sl] = cs - v + carry
      return carry + jnp.max(cs)

    lax.fori_loop(0, NBINS, scan, jnp.int32(0), unroll=4)

    def permute(t, _, shift=shift, src=src, dst=dst):
      pos = seg_base + t
      idxv = pos if src is None else plsc.load_gather(src, [pos])
      k = plsc.load_gather(key, [idxv])
      d = lax.shift_right_logical(k, shift) & 255
      addr = d * LANES + iota
      base = plsc.load_gather(hist, [addr])
      plsc.store_scatter(dst, [base], idxv)
      plsc.addupdate_scatter(hist, [addr], ones)
      return _

    lax.fori_loop(0, SEG, permute, None, unroll=UNROLL)

  def emit(t, _):
    sl = pl.ds(t * LANES, LANES)
    idxv = ia[sl]
    k = plsc.load_gather(key, [idxv])
    flip = (~lax.shift_right_arithmetic(k, 31)) | INT_MIN
    ib[sl] = k ^ flip
    return _

  lax.fori_loop(0, N // LANES, emit, None, unroll=UNROLL)


@functools.cache
def _make_sort_kernel():
  mesh = plsc.VectorSubcoreMesh(core_axis_name="c", subcore_axis_name="s")

  @functools.partial(
      pl.kernel,
      out_type=(
          jax.ShapeDtypeStruct((ROWS, N), jnp.int32),
          jax.ShapeDtypeStruct((ROWS, N), jnp.int32),
      ),
      mesh=mesh,
      compiler_params=pltpu.CompilerParams(needs_layout_passes=False),
      scratch_types=[
          pltpu.VMEM((N,), jnp.int32),
          pltpu.VMEM((N,), jnp.int32),
          pltpu.VMEM((N,), jnp.int32),
          pltpu.VMEM((NBINS * LANES,), jnp.int32),
      ],
  )
  def sort_kernel(x_hbm, vals_hbm, idx_hbm, key, ia, ib, hist):
    wid = lax.axis_index("s") * 2 + lax.axis_index("c")

    def do_row(i, _):
      r = wid * ROWS_PER_WORKER + i
      pltpu.sync_copy(x_hbm.at[r], key)
      _row_sort_body(key, ia, ib, hist)
      pltpu.sync_copy(ib, vals_hbm.at[r])
      pltpu.sync_copy(ia, idx_hbm.at[r])
      return _

    lax.fori_loop(0, ROWS_PER_WORKER, do_row, None)

  return sort_kernel


def kernel(x):
  bits = lax.bitcast_convert_type(x, jnp.int32)
  vals_bits, idx = _make_sort_kernel()(bits)
  vals = lax.bitcast_convert_type(vals_bits, jnp.float32)
  ok = jnp.array(True)
  return vals, idx, ok


# speedup vs baseline: 2.3802x; 1.5004x over previous
import functools

import jax
import jax.numpy as jnp
import numpy as np
from jax import lax
from jax.experimental import pallas as pl
from jax.experimental.pallas import tpu as pltpu
from jax.experimental.pallas import tpu_sc as plsc

ROWS = 64
N = 32768
LANES = 16
SEG = N // LANES
NBINS = 256
NPASS = 4
WORKERS = 32
ROWS_PER_WORKER = ROWS // WORKERS
INT_MIN = np.int32(-2147483648)
UNROLL = 8


def _row_sort_body(key, ia, ib, hist):
  iota = lax.iota(jnp.int32, LANES)
  seg_base = iota * SEG
  ones = jnp.full((LANES,), 1, jnp.int32)
  zeros = jnp.zeros((LANES,), jnp.int32)

  def zero_hist():
    @plsc.parallel_loop(0, NBINS, unroll=UNROLL)
    def _zero(j):
      hist[pl.ds(j * LANES, LANES)] = zeros

  zero_hist()

  @plsc.parallel_loop(0, SEG, unroll=UNROLL)
  def _xform_count(t):
    pos = seg_base + t
    v = plsc.load_gather(key, [pos])
    flip = lax.shift_right_arithmetic(v, 31) | INT_MIN
    k = v ^ flip
    plsc.store_scatter(key, [pos], k)
    d = k & 255
    plsc.addupdate_scatter(hist, [d * LANES + iota], ones)

  for p in range(NPASS):
    shift = 8 * p
    src = (ia, ib)[p % 2] if p > 0 else None
    dst = (ia, ib)[(p + 1) % 2]

    if p > 0:
      zero_hist()

      @plsc.parallel_loop(0, SEG, unroll=UNROLL)
      def _count(t, shift=shift, src=src):
        pos = seg_base + t
        idxv = plsc.load_gather(src, [pos])
        k = plsc.load_gather(key, [idxv])
        d = lax.shift_right_logical(k, shift) & 255
        plsc.addupdate_scatter(hist, [d * LANES + iota], ones)

    @plsc.parallel_loop(0, NBINS, unroll=4, carry=jnp.int32(0))
    def _scan(j, carry):
      sl = pl.ds(j * LANES, LANES)
      v = hist[sl]
      cs = plsc.cumsum(v)
      hist[sl] = cs - v + carry
      return carry + jnp.max(cs)

    def permute(t, _, shift=shift, src=src, dst=dst):
      pos = seg_base + t
      idxv = pos if src is None else plsc.load_gather(src, [pos])
      k = plsc.load_gather(key, [idxv])
      d = lax.shift_right_logical(k, shift) & 255
      addr = d * LANES + iota
      base = plsc.load_gather(hist, [addr])
      plsc.store_scatter(dst, [base], idxv)
      plsc.addupdate_scatter(hist, [addr], ones)
      return _

    lax.fori_loop(0, SEG, permute, None, unroll=UNROLL)

  @plsc.parallel_loop(0, N // LANES, unroll=UNROLL)
  def _emit(t):
    sl = pl.ds(t * LANES, LANES)
    idxv = ia[sl]
    k = plsc.load_gather(key, [idxv])
    flip = (~lax.shift_right_arithmetic(k, 31)) | INT_MIN
    ib[sl] = k ^ flip


@functools.cache
def _make_sort_kernel():
  mesh = plsc.VectorSubcoreMesh(core_axis_name="c", subcore_axis_name="s")

  @functools.partial(
      pl.kernel,
      out_type=(
          jax.ShapeDtypeStruct((ROWS, N), jnp.int32),
          jax.ShapeDtypeStruct((ROWS, N), jnp.int32),
      ),
      mesh=mesh,
      compiler_params=pltpu.CompilerParams(needs_layout_passes=False),
      scratch_types=[
          pltpu.VMEM((N,), jnp.int32),
          pltpu.VMEM((N,), jnp.int32),
          pltpu.VMEM((N,), jnp.int32),
          pltpu.VMEM((NBINS * LANES,), jnp.int32),
      ],
  )
  def sort_kernel(x_hbm, vals_hbm, idx_hbm, key, ia, ib, hist):
    wid = lax.axis_index("s") * 2 + lax.axis_index("c")

    def do_row(i, _):
      r = wid * ROWS_PER_WORKER + i
      pltpu.sync_copy(x_hbm.at[r], key)
      _row_sort_body(key, ia, ib, hist)
      pltpu.sync_copy(ib, vals_hbm.at[r])
      pltpu.sync_copy(ia, idx_hbm.at[r])
      return _

    lax.fori_loop(0, ROWS_PER_WORKER, do_row, None)

  return sort_kernel


def kernel(x):
  bits = lax.bitcast_convert_type(x, jnp.int32)
  vals_bits, idx = _make_sort_kernel()(bits)
  vals = lax.bitcast_convert_type(vals_bits, jnp.float32)
  ok = jnp.array(True)
  return vals, idx, ok


# speedup vs baseline: 3.3407x; 1.4035x over previous
import functools

import jax
import jax.numpy as jnp
import numpy as np
from jax import lax
from jax.experimental import pallas as pl
from jax.experimental.pallas import tpu as pltpu
from jax.experimental.pallas import tpu_sc as plsc

ROWS = 64
N = 32768
LANES = 16
VBATCH = 4
VL = LANES * VBATCH
SEGV = N // VL
NBINS = 256
NPASS = 4
WORKERS = 32
ROWS_PER_WORKER = ROWS // WORKERS
INT_MIN = np.int32(-2147483648)


def _row_sort_body(key, ia, ib, hist):
  iota = lax.iota(jnp.int32, LANES)
  pos_base = [iota * (VBATCH * SEGV) + j * SEGV for j in range(VBATCH)]
  vl_addr = [iota * VBATCH + j for j in range(VBATCH)]
  ones = jnp.full((LANES,), 1, jnp.int32)
  zeros = jnp.zeros((LANES,), jnp.int32)

  def zero_hist():
    @plsc.parallel_loop(0, NBINS * VL // LANES, unroll=8)
    def _zero(j):
      hist[pl.ds(j * LANES, LANES)] = zeros

  zero_hist()

  @plsc.parallel_loop(0, SEGV, unroll=2)
  def _xform_count(t):
    for j in range(VBATCH):
      pos = pos_base[j] + t
      v = plsc.load_gather(key, [pos])
      flip = lax.shift_right_arithmetic(v, 31) | INT_MIN
      k = v ^ flip
      plsc.store_scatter(key, [pos], k)
      d = k & 255
      plsc.addupdate_scatter(hist, [d * VL + vl_addr[j]], ones)

  for p in range(NPASS):
    shift = 8 * p
    src = (ia, ib)[p % 2] if p > 0 else None
    dst = (ia, ib)[(p + 1) % 2]

    if p > 0:
      zero_hist()

      @plsc.parallel_loop(0, SEGV, unroll=2)
      def _count(t, shift=shift, src=src):
        for j in range(VBATCH):
          pos = pos_base[j] + t
          idxv = plsc.load_gather(src, [pos])
          k = plsc.load_gather(key, [idxv])
          d = lax.shift_right_logical(k, shift) & 255
          plsc.addupdate_scatter(hist, [d * VL + vl_addr[j]], ones)

    @plsc.parallel_loop(0, NBINS * VL // LANES, unroll=4, carry=jnp.int32(0))
    def _scan(j, carry):
      sl = pl.ds(j * LANES, LANES)
      v = hist[sl]
      cs = plsc.cumsum(v)
      hist[sl] = cs - v + carry
      return carry + jnp.max(cs)

    def permute(t, _, shift=shift, src=src, dst=dst):
      idxs = []
      for j in range(VBATCH):
        pos = pos_base[j] + t
        idxs.append(pos if src is None else plsc.load_gather(src, [pos]))
      addrs = []
      for j in range(VBATCH):
        k = plsc.load_gather(key, [idxs[j]])
        d = lax.shift_right_logical(k, shift) & 255
        addrs.append(d * VL + vl_addr[j])
      bases = [plsc.load_gather(hist, [a]) for a in addrs]
      for j in range(VBATCH):
        plsc.store_scatter(dst, [bases[j]], idxs[j])
        plsc.addupdate_scatter(hist, [addrs[j]], ones)
      return _

    lax.fori_loop(0, SEGV, permute, None, unroll=2)

  @plsc.parallel_loop(0, N // LANES, unroll=8)
  def _emit(t):
    sl = pl.ds(t * LANES, LANES)
    idxv = ia[sl]
    k = plsc.load_gather(key, [idxv])
    flip = (~lax.shift_right_arithmetic(k, 31)) | INT_MIN
    ib[sl] = k ^ flip


@functools.cache
def _make_sort_kernel():
  mesh = plsc.VectorSubcoreMesh(core_axis_name="c", subcore_axis_name="s")

  @functools.partial(
      pl.kernel,
      out_type=(
          jax.ShapeDtypeStruct((ROWS, N), jnp.int32),
          jax.ShapeDtypeStruct((ROWS, N), jnp.int32),
      ),
      mesh=mesh,
      compiler_params=pltpu.CompilerParams(needs_layout_passes=False),
      scratch_types=[
          pltpu.VMEM((N,), jnp.int32),
          pltpu.VMEM((N,), jnp.int32),
          pltpu.VMEM((N,), jnp.int32),
          pltpu.VMEM((NBINS * VL,), jnp.int32),
      ],
  )
  def sort_kernel(x_hbm, vals_hbm, idx_hbm, key, ia, ib, hist):
    wid = lax.axis_index("s") * 2 + lax.axis_index("c")

    def do_row(i, _):
      r = wid * ROWS_PER_WORKER + i
      pltpu.sync_copy(x_hbm.at[r], key)
      _row_sort_body(key, ia, ib, hist)
      pltpu.sync_copy(ib, vals_hbm.at[r])
      pltpu.sync_copy(ia, idx_hbm.at[r])
      return _

    lax.fori_loop(0, ROWS_PER_WORKER, do_row, None)

  return sort_kernel


def kernel(x):
  bits = lax.bitcast_convert_type(x, jnp.int32)
  vals_bits, idx = _make_sort_kernel()(bits)
  vals = lax.bitcast_convert_type(vals_bits, jnp.float32)
  ok = jnp.array(True)
  return vals, idx, ok


# speedup vs baseline: 6.9824x; 2.0901x over previous
import functools

import jax
import jax.numpy as jnp
import numpy as np
from jax import lax
from jax.experimental import pallas as pl
from jax.experimental.pallas import tpu as pltpu
from jax.experimental.pallas import tpu_sc as plsc

ROWS = 64
N = 32768
LANES = 16
VBATCH = 4
VL = LANES * VBATCH
SEGV = N // VL
NBINS = 256
NPASS = 4
WORKERS = 32
ROWS_PER_WORKER = ROWS // WORKERS
INT_MIN = np.int32(-2147483648)
KEYPAD = N + N // SEGV


def _phi(idx):
  return idx + lax.shift_right_logical(idx, 9)


def _row_sort_body(key, ia, ib, hist):
  iota = lax.iota(jnp.int32, LANES)
  vl_addr = [jnp.int32(j * LANES) + iota for j in range(VBATCH)]
  q_base = [v * SEGV for v in vl_addr]
  qphi_base = [v * (SEGV + 1) for v in vl_addr]
  ones = jnp.full((LANES,), 1, jnp.int32)
  zeros = jnp.zeros((LANES,), jnp.int32)

  def zero_hist():
    @plsc.parallel_loop(0, NBINS * VL // LANES, unroll=8)
    def _zero(j):
      hist[pl.ds(j * LANES, LANES)] = zeros

  zero_hist()

  @plsc.parallel_loop(0, N // LANES, unroll=8)
  def _xform(t):
    pos = t * LANES + iota
    v = ib[pl.ds(t * LANES, LANES)]
    flip = lax.shift_right_arithmetic(v, 31) | INT_MIN
    plsc.store_scatter(key, [_phi(pos)], v ^ flip)

  @plsc.parallel_loop(0, SEGV, unroll=2)
  def _count0(t):
    for j in range(VBATCH):
      k = plsc.load_gather(key, [qphi_base[j] + t])
      d = k & 255
      plsc.addupdate_scatter(hist, [d * VL + vl_addr[j]], ones)

  for p in range(NPASS):
    shift = 8 * p
    src = (ib, ia)[p % 2] if p > 0 else None
    dst = (ib, ia)[(p + 1) % 2]
    last = p == NPASS - 1

    if p > 0:
      zero_hist()

      @plsc.parallel_loop(0, SEGV, unroll=2)
      def _count(t, shift=shift, src=src):
        for j in range(VBATCH):
          idxv = src[pl.ds(t * VL + j * LANES, LANES)]
          k = plsc.load_gather(key, [_phi(idxv)])
          d = lax.shift_right_logical(k, shift) & 255
          plsc.addupdate_scatter(hist, [d * VL + vl_addr[j]], ones)

    @plsc.parallel_loop(0, NBINS * VL // LANES, unroll=4, carry=jnp.int32(0))
    def _scan(j, carry):
      sl = pl.ds(j * LANES, LANES)
      v = hist[sl]
      cs = plsc.cumsum(v)
      hist[sl] = cs - v + carry
      return carry + jnp.max(cs)

    def permute(t, _, shift=shift, src=src, dst=dst, last=last):
      idxs = []
      for j in range(VBATCH):
        if src is None:
          idxs.append(q_base[j] + t)
        else:
          idxs.append(src[pl.ds(t * VL + j * LANES, LANES)])
      addrs = []
      for j in range(VBATCH):
        k = plsc.load_gather(key, [_phi(idxs[j])])
        d = lax.shift_right_logical(k, shift) & 255
        addrs.append(d * VL + vl_addr[j])
      bases = [plsc.load_gather(hist, [a]) for a in addrs]
      for j in range(VBATCH):
        base = bases[j]
        if last:
          wpos = base
        else:
          wpos = ((base & (SEGV - 1)) * VL) | lax.shift_right_logical(base, 9)
        plsc.store_scatter(dst, [wpos], idxs[j])
        plsc.addupdate_scatter(hist, [addrs[j]], ones)
      return _

    lax.fori_loop(0, SEGV, permute, None)

  @plsc.parallel_loop(0, N // LANES, unroll=8)
  def _emit(t):
    sl = pl.ds(t * LANES, LANES)
    idxv = ib[sl]
    k = plsc.load_gather(key, [_phi(idxv)])
    flip = (~lax.shift_right_arithmetic(k, 31)) | INT_MIN
    ia[sl] = k ^ flip


@functools.cache
def _make_sort_kernel():
  mesh = plsc.VectorSubcoreMesh(core_axis_name="c", subcore_axis_name="s")

  @functools.partial(
      pl.kernel,
      out_type=(
          jax.ShapeDtypeStruct((ROWS, N), jnp.int32),
          jax.ShapeDtypeStruct((ROWS, N), jnp.int32),
      ),
      mesh=mesh,
      compiler_params=pltpu.CompilerParams(needs_layout_passes=False),
      scratch_types=[
          pltpu.VMEM((KEYPAD,), jnp.int32),
          pltpu.VMEM((N,), jnp.int32),
          pltpu.VMEM((N,), jnp.int32),
          pltpu.VMEM((NBINS * VL,), jnp.int32),
      ],
  )
  def sort_kernel(x_hbm, vals_hbm, idx_hbm, key, ia, ib, hist):
    wid = lax.axis_index("s") * 2 + lax.axis_index("c")

    def do_row(i, _):
      r = wid * ROWS_PER_WORKER + i
      pltpu.sync_copy(x_hbm.at[r], ib)
      _row_sort_body(key, ia, ib, hist)
      pltpu.sync_copy(ia, vals_hbm.at[r])
      pltpu.sync_copy(ib, idx_hbm.at[r])
      return _

    lax.fori_loop(0, ROWS_PER_WORKER, do_row, None)

  return sort_kernel


def kernel(x):
  bits = lax.bitcast_convert_type(x, jnp.int32)
  vals_bits, idx = _make_sort_kernel()(bits)
  vals = lax.bitcast_convert_type(vals_bits, jnp.float32)
  ok = jnp.array(True)
  return vals, idx, ok


# speedup vs baseline: 8.1771x; 1.1711x over previous
import functools

import jax
import jax.numpy as jnp
import numpy as np
from jax import lax
from jax.experimental import pallas as pl
from jax.experimental.pallas import tpu as pltpu
from jax.experimental.pallas import tpu_sc as plsc

ROWS = 64
N = 32768
LANES = 16
VBATCH = 4
VL = LANES * VBATCH
SEGV = N // VL
NBINS = 256
NPASS = 4
WORKERS = 32
ROWS_PER_WORKER = ROWS // WORKERS
INT_MIN = np.int32(-2147483648)
KEYPAD = N + N // SEGV


def _phi(idx):
  return idx + lax.shift_right_logical(idx, 9)


def _row_sort_body(key, ia, ib, hist):
  iota = lax.iota(jnp.int32, LANES)
  vl_addr = [jnp.int32(j * LANES) + iota for j in range(VBATCH)]
  q_base = [v * SEGV for v in vl_addr]
  qphi_base = [v * (SEGV + 1) for v in vl_addr]
  ones = jnp.full((LANES,), 1, jnp.int32)
  zeros = jnp.zeros((LANES,), jnp.int32)

  def zero_hist():
    @plsc.parallel_loop(0, NBINS * VL // LANES, unroll=8)
    def _zero(j):
      hist[pl.ds(j * LANES, LANES)] = zeros

  zero_hist()

  @plsc.parallel_loop(0, N // LANES, unroll=8)
  def _xform(t):
    pos = t * LANES + iota
    v = ib[pl.ds(t * LANES, LANES)]
    flip = lax.shift_right_arithmetic(v, 31) | INT_MIN
    plsc.store_scatter(key, [_phi(pos)], v ^ flip)

  for p in range(NPASS):
    shift = 8 * p
    src = (ib, ia)[p % 2]
    dst = (ib, ia)[(p + 1) % 2]
    last = p == NPASS - 1

    if p > 0:
      zero_hist()

    @plsc.parallel_loop(0, SEGV, unroll=2)
    def _count(t, shift=shift, src=src, p=p):
      for j in range(VBATCH):
        sl = pl.ds(t * VL + j * LANES, LANES)
        idxv = (q_base[j] + t) if p == 0 else src[sl]
        k = plsc.load_gather(key, [(qphi_base[j] + t) if p == 0 else _phi(idxv)])
        d = lax.shift_right_logical(k, shift) & 255
        addr = d * VL + vl_addr[j]
        plsc.addupdate_scatter(hist, [addr], ones)
        src[sl] = (addr << 17) | idxv

    @plsc.parallel_loop(0, NBINS * VL // LANES, unroll=4, carry=jnp.int32(0))
    def _scan(j, carry):
      sl = pl.ds(j * LANES, LANES)
      v = hist[sl]
      cs = plsc.cumsum(v)
      hist[sl] = cs - v + carry
      return carry + jnp.max(cs)

    def permute(t, _, src=src, dst=dst, last=last):
      packed = [src[pl.ds(t * VL + j * LANES, LANES)] for j in range(VBATCH)]
      addrs = [lax.shift_right_logical(v, 17) for v in packed]
      bases = [plsc.load_gather(hist, [a]) for a in addrs]
      for j in range(VBATCH):
        base = bases[j]
        if last:
          wpos = base
        else:
          wpos = ((base & (SEGV - 1)) * VL) | lax.shift_right_logical(base, 9)
        plsc.store_scatter(dst, [wpos], packed[j] & 131071)
        plsc.addupdate_scatter(hist, [addrs[j]], ones)
      return _

    lax.fori_loop(0, SEGV, permute, None, unroll=2)

  @plsc.parallel_loop(0, N // LANES, unroll=8)
  def _emit(t):
    sl = pl.ds(t * LANES, LANES)
    idxv = ib[sl]
    k = plsc.load_gather(key, [_phi(idxv)])
    flip = (~lax.shift_right_arithmetic(k, 31)) | INT_MIN
    ia[sl] = k ^ flip


@functools.cache
def _make_sort_kernel():
  mesh = plsc.VectorSubcoreMesh(core_axis_name="c", subcore_axis_name="s")

  @functools.partial(
      pl.kernel,
      out_type=(
          jax.ShapeDtypeStruct((ROWS, N), jnp.int32),
          jax.ShapeDtypeStruct((ROWS, N), jnp.int32),
      ),
      mesh=mesh,
      compiler_params=pltpu.CompilerParams(needs_layout_passes=False),
      scratch_types=[
          pltpu.VMEM((KEYPAD,), jnp.int32),
          pltpu.VMEM((N,), jnp.int32),
          pltpu.VMEM((N,), jnp.int32),
          pltpu.VMEM((NBINS * VL,), jnp.int32),
      ],
  )
  def sort_kernel(x_hbm, vals_hbm, idx_hbm, key, ia, ib, hist):
    wid = lax.axis_index("s") * 2 + lax.axis_index("c")

    def do_row(i, _):
      r = wid * ROWS_PER_WORKER + i
      pltpu.sync_copy(x_hbm.at[r], ib)
      _row_sort_body(key, ia, ib, hist)
      pltpu.sync_copy(ia, vals_hbm.at[r])
      pltpu.sync_copy(ib, idx_hbm.at[r])
      return _

    lax.fori_loop(0, ROWS_PER_WORKER, do_row, None)

  return sort_kernel


def kernel(x):
  bits = lax.bitcast_convert_type(x, jnp.int32)
  vals_bits, idx = _make_sort_kernel()(bits)
  vals = lax.bitcast_convert_type(vals_bits, jnp.float32)
  ok = jnp.array(True)
  return vals, idx, ok


# speedup vs baseline: 8.4030x; 1.0276x over previous
import functools

import jax
import jax.numpy as jnp
import numpy as np
from jax import lax
from jax.experimental import pallas as pl
from jax.experimental.pallas import tpu as pltpu
from jax.experimental.pallas import tpu_sc as plsc

ROWS = 64
N = 32768
LANES = 16
VBATCH = 4
VL = LANES * VBATCH
SEGV = N // VL
NBINS = 256
NPASS = 4
WORKERS = 32
ROWS_PER_WORKER = ROWS // WORKERS
INT_MIN = np.int32(-2147483648)
KEYPAD = N + N // SEGV


def _phi(idx):
  return idx + lax.shift_right_logical(idx, 9)


def _row_sort_body(key, ia, ib, hist):
  iota = lax.iota(jnp.int32, LANES)
  vl_addr = [jnp.int32(j * LANES) + iota for j in range(VBATCH)]
  q_base = [v * SEGV for v in vl_addr]
  qphi_base = [v * (SEGV + 1) for v in vl_addr]
  ones = jnp.full((LANES,), 1, jnp.int32)
  zeros = jnp.zeros((LANES,), jnp.int32)

  def zero_hist():
    @plsc.parallel_loop(0, NBINS * VL // LANES, unroll=8)
    def _zero(j):
      hist[pl.ds(j * LANES, LANES)] = zeros

  zero_hist()

  @plsc.parallel_loop(0, N // LANES, unroll=8)
  def _xform(t):
    pos = t * LANES + iota
    v = ib[pl.ds(t * LANES, LANES)]
    flip = lax.shift_right_arithmetic(v, 31) | INT_MIN
    plsc.store_scatter(key, [_phi(pos)], v ^ flip)

  for p in range(NPASS):
    shift = 8 * p
    src = (ib, ia)[p % 2]
    dst = (ib, ia)[(p + 1) % 2]
    last = p == NPASS - 1

    if p > 0:
      zero_hist()

    @plsc.parallel_loop(0, SEGV, unroll=2)
    def _count(t, shift=shift, src=src, p=p):
      for j in range(VBATCH):
        sl = pl.ds(t * VL + j * LANES, LANES)
        idxv = (q_base[j] + t) if p == 0 else src[sl]
        k = plsc.load_gather(key, [(qphi_base[j] + t) if p == 0 else _phi(idxv)])
        d = lax.shift_right_logical(k, shift) & 255
        addr = d * VL + vl_addr[j]
        plsc.addupdate_scatter(hist, [addr], ones)
        src[sl] = (addr << 17) | idxv

    @plsc.parallel_loop(0, NBINS * VL // LANES, unroll=4, carry=jnp.int32(0))
    def _scan(j, carry):
      sl = pl.ds(j * LANES, LANES)
      v = hist[sl]
      cs = plsc.cumsum(v)
      hist[sl] = cs - v + carry
      return carry + jnp.max(cs)

    def permute(t, _, src=src, dst=dst, last=last):
      packed = [src[pl.ds(t * VL + j * LANES, LANES)] for j in range(VBATCH)]
      addrs = [lax.shift_right_logical(v, 17) for v in packed]
      bases = [plsc.load_gather(hist, [a]) for a in addrs]
      for j in range(VBATCH):
        plsc.store_scatter(hist, [addrs[j]], bases[j] + ones)
      for j in range(VBATCH):
        base = bases[j]
        if last:
          wpos = base
        else:
          wpos = ((base & (SEGV - 1)) * VL) | lax.shift_right_logical(base, 9)
        plsc.store_scatter(dst, [wpos], packed[j] & 131071)
      return _

    lax.fori_loop(0, SEGV, permute, None, unroll=2)

  @plsc.parallel_loop(0, N // LANES, unroll=8)
  def _emit(t):
    sl = pl.ds(t * LANES, LANES)
    idxv = ib[sl]
    k = plsc.load_gather(key, [_phi(idxv)])
    flip = (~lax.shift_right_arithmetic(k, 31)) | INT_MIN
    ia[sl] = k ^ flip


@functools.cache
def _make_sort_kernel():
  mesh = plsc.VectorSubcoreMesh(core_axis_name="c", subcore_axis_name="s")

  @functools.partial(
      pl.kernel,
      out_type=(
          jax.ShapeDtypeStruct((ROWS, N), jnp.int32),
          jax.ShapeDtypeStruct((ROWS, N), jnp.int32),
      ),
      mesh=mesh,
      compiler_params=pltpu.CompilerParams(needs_layout_passes=False),
      scratch_types=[
          pltpu.VMEM((KEYPAD,), jnp.int32),
          pltpu.VMEM((N,), jnp.int32),
          pltpu.VMEM((N,), jnp.int32),
          pltpu.VMEM((NBINS * VL,), jnp.int32),
      ],
  )
  def sort_kernel(x_hbm, vals_hbm, idx_hbm, key, ia, ib, hist):
    wid = lax.axis_index("s") * 2 + lax.axis_index("c")

    def do_row(i, _):
      r = wid * ROWS_PER_WORKER + i
      pltpu.sync_copy(x_hbm.at[r], ib)
      _row_sort_body(key, ia, ib, hist)
      pltpu.sync_copy(ia, vals_hbm.at[r])
      pltpu.sync_copy(ib, idx_hbm.at[r])
      return _

    lax.fori_loop(0, ROWS_PER_WORKER, do_row, None)

  return sort_kernel


def kernel(x):
  bits = lax.bitcast_convert_type(x, jnp.int32)
  vals_bits, idx = _make_sort_kernel()(bits)
  vals = lax.bitcast_convert_type(vals_bits, jnp.float32)
  ok = jnp.array(True)
  return vals, idx, ok
